# baseline (speedup 1.0000x reference)
import jax
import jax.numpy as jnp
from jax.experimental import pallas as pl


_BS = 256


def _add_body(x_ref, pe_ref, o_ref):
    o_ref[...] = x_ref[...] + pe_ref[...][None, :, :]


def kernel(x, pe_table):
    batch, seq, d = x.shape
    num_blocks = seq // _BS
    return pl.pallas_call(
        _add_body,
        grid=(num_blocks,),
        in_specs=[
            pl.BlockSpec((batch, _BS, d), lambda i: (0, i, 0)),
            pl.BlockSpec((_BS, d), lambda i: (i, 0)),
        ],
        out_specs=pl.BlockSpec((batch, _BS, d), lambda i: (0, i, 0)),
        out_shape=jax.ShapeDtypeStruct(x.shape, x.dtype),
    )(x, pe_table)


# speedup vs baseline: 1.0255x; 1.0255x over previous
import jax
import jax.numpy as jnp
from jax.experimental import pallas as pl


_BS = 512


def _add_body(x_ref, pe_ref, o_ref):
    o_ref[...] = x_ref[...] + pe_ref[...][None, :, :]


def kernel(x, pe_table):
    batch, seq, d = x.shape
    num_blocks = seq // _BS
    return pl.pallas_call(
        _add_body,
        grid=(num_blocks,),
        in_specs=[
            pl.BlockSpec((batch, _BS, d), lambda i: (0, i, 0)),
            pl.BlockSpec((_BS, d), lambda i: (i, 0)),
        ],
        out_specs=pl.BlockSpec((batch, _BS, d), lambda i: (0, i, 0)),
        out_shape=jax.ShapeDtypeStruct(x.shape, x.dtype),
    )(x, pe_table)
